# baseline (speedup 1.0000x reference)
import functools

import jax
import jax.numpy as jnp
from jax import lax
from jax.experimental import pallas as pl
from jax.experimental.pallas import tpu as pltpu
from jax.experimental.pallas import tpu_sc as plsc

B = 8
T = 4096
D = 512
TT = 4095
NT = 1365
NW = 32
CTR = 3
NCHUNK = NT // CTR
FULL_PAIRS = 7
TBASE = NW * 2 * FULL_PAIRS * CTR
KR = CTR * B
GROUPS = (0, 8)

_mesh = plsc.VectorSubcoreMesh(
    core_axis_name="c", subcore_axis_name="s", num_cores=2, num_subcores=16
)


@functools.partial(
    pl.kernel,
    mesh=_mesh,
    out_type=jax.ShapeDtypeStruct((NT, B, 5 * D), jnp.float32),
    scratch_types=[
        pltpu.VMEM((5, KR), jnp.int32),
        pltpu.VMEM((5, KR), jnp.int32),
        pltpu.VMEM((KR, 5 * D), jnp.float32),
        pltpu.VMEM((KR, 5 * D), jnp.float32),
        pltpu.SemaphoreType.DMA,
        pltpu.SemaphoreType.DMA,
        pltpu.SemaphoreType.DMA,
        pltpu.SemaphoreType.DMA,
    ],
)
def _splice_gather(feats_hbm, out_hbm, idx0_v, idx1_v, rows0_v, rows1_v,
                   sem_g0, sem_g1, sem_o0, sem_o1):
    wid = lax.axis_index("s") * 2 + lax.axis_index("c")
    lanes = lax.iota(jnp.int32, 16)

    pats = []
    for off in GROUPS:
        pos = off + lanes
        tloc = lax.shift_right_logical(pos, 3)
        bpat = (pos - tloc * B) * T
        pats.append((3 * tloc, bpat))

    def fill_idx(idx_v, cc):
        s0 = cc * (3 * CTR)
        for (t3, bpat), off in zip(pats, GROUPS):
            for kk in range(5):
                idx_v[kk, pl.ds(off, 16)] = (
                    bpat + jnp.clip(s0 + t3 + (kk - 2), 0, TT - 1))

    def start_gather(idx_v, rows_v, sem, cc):
        fill_idx(idx_v, cc)
        for kk in range(5):
            pltpu.async_copy(feats_hbm.at[idx_v.at[kk]],
                             rows_v.at[:, pl.ds(kk * D, D)], sem)

    def wait_gather(idx_v, rows_v, sem):
        for kk in range(5):
            pltpu.make_async_copy(feats_hbm.at[idx_v.at[kk]],
                                  rows_v.at[:, pl.ds(kk * D, D)], sem).wait()

    def start_put(rows_v, sem, cc):
        pltpu.async_copy(rows_v.reshape(CTR, B, 5 * D),
                         out_hbm.at[pl.ds(cc * CTR, CTR)], sem)

    def wait_put(rows_v, sem):
        pltpu.make_async_copy(rows_v.reshape(CTR, B, 5 * D),
                              out_hbm.at[pl.ds(0, CTR)], sem).wait()

    def chunk(j):
        return wid + j * NW

    start_gather(idx0_v, rows0_v, sem_g0, chunk(0))

    @pl.loop(0, FULL_PAIRS)
    def _pair(m):
        @pl.when(m > 0)
        def _():
            wait_put(rows1_v, sem_o1)

        start_gather(idx1_v, rows1_v, sem_g1, chunk(2 * m + 1))
        wait_gather(idx0_v, rows0_v, sem_g0)
        start_put(rows0_v, sem_o0, chunk(2 * m))
        wait_gather(idx1_v, rows1_v, sem_g1)
        start_put(rows1_v, sem_o1, chunk(2 * m + 1))

        @pl.when(m < FULL_PAIRS - 1)
        def _():
            wait_put(rows0_v, sem_o0)
            start_gather(idx0_v, rows0_v, sem_g0, chunk(2 * m + 2))

    wait_put(rows1_v, sem_o1)
    wait_put(rows0_v, sem_o0)

    @pl.when(wid < NT - TBASE)
    def _():
        tr = TBASE + wid
        for kk in range(5):
            idx0_v[kk, pl.ds(0, 16)] = (
                lanes * T + jnp.clip(3 * tr + (kk - 2), 0, TT - 1))
        for kk in range(5):
            pltpu.async_copy(feats_hbm.at[idx0_v.at[kk, pl.ds(0, B)]],
                             rows0_v.at[pl.ds(0, B), pl.ds(kk * D, D)],
                             sem_g0)
        for kk in range(5):
            pltpu.make_async_copy(feats_hbm.at[idx0_v.at[kk, pl.ds(0, B)]],
                                  rows0_v.at[pl.ds(0, B), pl.ds(kk * D, D)],
                                  sem_g0).wait()
        pltpu.async_copy(rows0_v.at[pl.ds(0, B)], out_hbm.at[tr], sem_o0)
        pltpu.make_async_copy(rows0_v.at[pl.ds(0, B)], out_hbm.at[tr],
                              sem_o0).wait()


def kernel(feats):
    out = _splice_gather(feats.reshape(B * T, D))
    return out.transpose(1, 0, 2)


# speedup vs baseline: 1.0205x; 1.0205x over previous
import functools

import jax
import jax.numpy as jnp
from jax import lax
from jax.experimental import pallas as pl
from jax.experimental.pallas import tpu as pltpu
from jax.experimental.pallas import tpu_sc as plsc

B = 8
T = 4096
D = 512
TT = 4095
NT = 1365
NW = 32
CTR = 3
NCHUNK = NT // CTR
FULL_PAIRS = 7
TBASE = NW * 2 * FULL_PAIRS * CTR
KR = CTR * B
GROUPS = (0, 8)

_mesh = plsc.VectorSubcoreMesh(
    core_axis_name="c", subcore_axis_name="s", num_cores=2, num_subcores=16
)


@functools.partial(
    pl.kernel,
    mesh=_mesh,
    out_type=jax.ShapeDtypeStruct((NT, B, 5 * D), jnp.float32),
    scratch_types=[
        pltpu.VMEM((5, KR), jnp.int32),
        pltpu.VMEM((5, KR), jnp.int32),
        pltpu.VMEM((KR, 5 * D), jnp.float32),
        pltpu.VMEM((KR, 5 * D), jnp.float32),
        pltpu.SemaphoreType.DMA,
        pltpu.SemaphoreType.DMA,
        pltpu.SemaphoreType.DMA,
        pltpu.SemaphoreType.DMA,
    ],
)
def _splice_gather(feats_hbm, out_hbm, idx0_v, idx1_v, rows0_v, rows1_v,
                   sem_g0, sem_g1, sem_o0, sem_o1):
    wid = lax.axis_index("s") * 2 + lax.axis_index("c")
    lanes = lax.iota(jnp.int32, 16)

    pats = []
    for off in GROUPS:
        pos = off + lanes
        tloc = lax.shift_right_logical(pos, 3)
        bpat = (pos - tloc * B) * T
        pats.append((3 * tloc, bpat))

    def fill_idx(idx_v, cc):
        s0 = cc * (3 * CTR)
        for (t3, bpat), off in zip(pats, GROUPS):
            for kk in range(5):
                idx_v[kk, pl.ds(off, 16)] = (
                    bpat + jnp.clip(s0 + t3 + (kk - 2), 0, TT - 1))

    def start_gather(idx_v, rows_v, sem, cc):
        fill_idx(idx_v, cc)
        for kk in range(5):
            pltpu.async_copy(feats_hbm.at[idx_v.at[kk]],
                             rows_v.at[:, pl.ds(kk * D, D)], sem)

    def wait_gather(idx_v, rows_v, sem):
        for kk in range(5):
            pltpu.make_async_copy(feats_hbm.at[idx_v.at[kk]],
                                  rows_v.at[:, pl.ds(kk * D, D)], sem).wait()

    def start_put(rows_v, sem, cc):
        pltpu.async_copy(rows_v.reshape(CTR, B, 5 * D),
                         out_hbm.at[pl.ds(cc * CTR, CTR)], sem)

    def wait_put(rows_v, sem):
        pltpu.make_async_copy(rows_v.reshape(CTR, B, 5 * D),
                              out_hbm.at[pl.ds(0, CTR)], sem).wait()

    def chunk(j):
        return wid + j * NW

    start_gather(idx0_v, rows0_v, sem_g0, chunk(0))

    @pl.loop(0, FULL_PAIRS)
    def _pair(m):
        @pl.when(m > 0)
        def _():
            wait_put(rows1_v, sem_o1)

        start_gather(idx1_v, rows1_v, sem_g1, chunk(2 * m + 1))
        wait_gather(idx0_v, rows0_v, sem_g0)
        start_put(rows0_v, sem_o0, chunk(2 * m))

        @pl.when(m < FULL_PAIRS - 1)
        def _():
            wait_put(rows0_v, sem_o0)
            start_gather(idx0_v, rows0_v, sem_g0, chunk(2 * m + 2))

        wait_gather(idx1_v, rows1_v, sem_g1)
        start_put(rows1_v, sem_o1, chunk(2 * m + 1))

    wait_put(rows1_v, sem_o1)
    wait_put(rows0_v, sem_o0)

    @pl.when(wid < NT - TBASE)
    def _():
        tr = TBASE + wid
        for kk in range(5):
            idx0_v[kk, pl.ds(0, 16)] = (
                lanes * T + jnp.clip(3 * tr + (kk - 2), 0, TT - 1))
        for kk in range(5):
            pltpu.async_copy(feats_hbm.at[idx0_v.at[kk, pl.ds(0, B)]],
                             rows0_v.at[pl.ds(0, B), pl.ds(kk * D, D)],
                             sem_g0)
        for kk in range(5):
            pltpu.make_async_copy(feats_hbm.at[idx0_v.at[kk, pl.ds(0, B)]],
                                  rows0_v.at[pl.ds(0, B), pl.ds(kk * D, D)],
                                  sem_g0).wait()
        pltpu.async_copy(rows0_v.at[pl.ds(0, B)], out_hbm.at[tr], sem_o0)
        pltpu.make_async_copy(rows0_v.at[pl.ds(0, B)], out_hbm.at[tr],
                              sem_o0).wait()


def kernel(feats):
    out = _splice_gather(feats.reshape(B * T, D))
    return out.transpose(1, 0, 2)
